# baseline (speedup 1.0000x reference)
import jax
import jax.numpy as jnp
import numpy as np
from jax.experimental import pallas as pl

_B = 4096
_ZDIM = 128
_COND = 100
_HID = 256
_N_CONT = 20
_MODES = 10
_N_DISC = 10
_CATS = 10
_PAC = 8
_TAU = 0.2
_ROW = 320
_NG = _N_CONT + _N_DISC
_GRP = _MODES * _NG
_BLK = 1024

_HIGH = jax.lax.Precision.HIGHEST


def _perm_matrix() -> np.ndarray:
    dst = np.zeros((_ROW,), np.int64)
    for s in range(_N_CONT):
        dst[s] = 11 * s
    for m in range(_MODES):
        for g in range(_NG):
            s = _N_CONT + _NG * m + g
            if g < _N_CONT:
                dst[s] = 11 * g + 1 + m
            else:
                dst[s] = 220 + 10 * (g - _N_CONT) + m
    P = np.zeros((_ROW, _ROW), np.float32)
    P[np.arange(_ROW), dst] = 1.0
    return P


_PERM_NP = _perm_matrix()


def _gen_body(z_ref, c_ref, w1z_ref, w1c_ref, b1_ref, g1_ref, be1_ref,
              w2_ref, b2_ref, g2_ref, be2_ref,
              wall_ref, ball_ref, g_ref, p_ref,
              wc1_ref, bc1_ref, wc2_ref, bc2_ref, wo_ref, bo_ref,
              out_ref, sc_ref):
    isq = 1.0 / jnp.sqrt(jnp.float32(1.0 + 1e-3))
    h1 = jnp.dot(z_ref[...], w1z_ref[...])
    h1 = h1 + jnp.dot(c_ref[...], w1c_ref[...])
    h1 = g1_ref[...] * ((h1 + b1_ref[...]) * isq) + be1_ref[...]
    h1 = jnp.maximum(h1, 0.0)
    h2 = jnp.dot(h1, w2_ref[...])
    h2 = g2_ref[...] * ((h2 + b2_ref[...]) * isq) + be2_ref[...]
    h2 = jnp.maximum(h2, 0.0)
    t = jnp.dot(h2, wall_ref[...]) + ball_ref[...]
    a = jnp.tanh(t[:, :_N_CONT])
    s = (t[:, _N_CONT:] + g_ref[...]) / _TAU
    sl = [s[:, k * _NG:(k + 1) * _NG] for k in range(_MODES)]
    m = sl[0]
    for k in range(1, _MODES):
        m = jnp.maximum(m, sl[k])
    es = [jnp.exp(v - m) for v in sl]
    den = es[0]
    for k in range(1, _MODES):
        den = den + es[k]
    ys = [e / den for e in es]
    ymax = ys[0]
    for k in range(1, _MODES):
        ymax = jnp.maximum(ymax, ys[k])
    outs = [((y == ymax).astype(jnp.float32) - y) + y for y in ys]
    u = jnp.concatenate([a] + outs, axis=1)
    fr = jnp.dot(u, p_ref[...], precision=_HIGH)
    out_ref[...] = fr
    fr3 = fr.reshape(_BLK // _PAC, _PAC, _ROW)
    hc = jnp.dot(fr3[:, 0, :], wc1_ref[0:_ROW, :], precision=_HIGH)
    for p in range(1, _PAC):
        hc = hc + jnp.dot(fr3[:, p, :], wc1_ref[p * _ROW:(p + 1) * _ROW, :],
                          precision=_HIGH)
    hc = hc + bc1_ref[...]
    hc = jnp.where(hc >= 0, hc, 0.2 * hc)
    hc = jnp.dot(hc, wc2_ref[...], precision=_HIGH) + bc2_ref[...]
    hc = jnp.where(hc >= 0, hc, 0.2 * hc)
    sc_ref[...] = jnp.dot(hc, wo_ref[...], precision=_HIGH) + bo_ref[...]


def _critic_body(x_ref, w1_ref, b1_ref, w2_ref, b2_ref, wo_ref, bo_ref, out_ref):
    h = jnp.dot(x_ref[...], w1_ref[...], precision=_HIGH) + b1_ref[...]
    h = jnp.where(h >= 0, h, 0.2 * h)
    h = jnp.dot(h, w2_ref[...], precision=_HIGH) + b2_ref[...]
    h = jnp.where(h >= 0, h, 0.2 * h)
    out_ref[...] = jnp.dot(h, wo_ref[...], precision=_HIGH) + bo_ref[...]


def _gumbel_noise() -> jnp.ndarray:
    gkey = jax.random.key(42)

    def draw(k):
        U = jax.random.uniform(k, (_B, _MODES), minval=1e-6, maxval=1.0 - 1e-6)
        return -jnp.log(-jnp.log(U + 1e-20) + 1e-20)

    kc = jax.vmap(lambda i: jax.random.fold_in(gkey, i))(jnp.arange(_N_CONT))
    kd = jax.vmap(lambda i: jax.random.fold_in(gkey, i))(jnp.arange(1000, 1000 + _N_DISC))
    gc = jax.vmap(draw)(kc)
    gd = jax.vmap(draw)(kd)
    g = jnp.concatenate([jnp.transpose(gc, (1, 2, 0)),
                         jnp.transpose(gd, (1, 2, 0))], axis=2)
    return g.reshape(_B, _GRP)


def kernel(z, cond, W1, b1, g1, be1, W2, b2, g2, be2, Wa, ba, Wb, bb, Wd, bd,
           Wc1, bc1, Wc2, bc2, Wo, bo):
    wa = jnp.transpose(jnp.squeeze(Wa, -1))
    wg = jnp.concatenate([jnp.transpose(Wb, (1, 2, 0)),
                          jnp.transpose(Wd, (1, 2, 0))], axis=2)
    wall = jnp.concatenate([wa, wg.reshape(_HID, _GRP)], axis=1)
    bgrp = jnp.concatenate([jnp.transpose(bb), jnp.transpose(bd)], axis=1).reshape(_GRP)
    ball = jnp.concatenate([jnp.squeeze(ba, -1), bgrp])[None, :]
    gfull = _gumbel_noise()
    pmat = jnp.asarray(_PERM_NP)

    nblk = _B // _BLK
    full_row, score = pl.pallas_call(
        _gen_body,
        grid=(nblk,),
        in_specs=[
            pl.BlockSpec((_BLK, _ZDIM), lambda i: (i, 0)),
            pl.BlockSpec((_BLK, _COND), lambda i: (i, 0)),
            pl.BlockSpec((_ZDIM, _HID), lambda i: (0, 0)),
            pl.BlockSpec((_COND, _HID), lambda i: (0, 0)),
            pl.BlockSpec((1, _HID), lambda i: (0, 0)),
            pl.BlockSpec((1, _HID), lambda i: (0, 0)),
            pl.BlockSpec((1, _HID), lambda i: (0, 0)),
            pl.BlockSpec((_HID, _HID), lambda i: (0, 0)),
            pl.BlockSpec((1, _HID), lambda i: (0, 0)),
            pl.BlockSpec((1, _HID), lambda i: (0, 0)),
            pl.BlockSpec((1, _HID), lambda i: (0, 0)),
            pl.BlockSpec((_HID, _ROW), lambda i: (0, 0)),
            pl.BlockSpec((1, _ROW), lambda i: (0, 0)),
            pl.BlockSpec((_BLK, _GRP), lambda i: (i, 0)),
            pl.BlockSpec((_ROW, _ROW), lambda i: (0, 0)),
            pl.BlockSpec((_PAC * _ROW, _HID), lambda i: (0, 0)),
            pl.BlockSpec((1, _HID), lambda i: (0, 0)),
            pl.BlockSpec((_HID, _HID), lambda i: (0, 0)),
            pl.BlockSpec((1, _HID), lambda i: (0, 0)),
            pl.BlockSpec((_HID, 1), lambda i: (0, 0)),
            pl.BlockSpec((1, 1), lambda i: (0, 0)),
        ],
        out_specs=[
            pl.BlockSpec((_BLK, _ROW), lambda i: (i, 0)),
            pl.BlockSpec((_BLK // _PAC, 1), lambda i: (i, 0)),
        ],
        out_shape=[
            jax.ShapeDtypeStruct((_B, _ROW), jnp.float32),
            jax.ShapeDtypeStruct((_B // _PAC, 1), jnp.float32),
        ],
    )(z, cond, W1[:_ZDIM], W1[_ZDIM:], b1[None, :], g1[None, :], be1[None, :],
      W2, b2[None, :], g2[None, :], be2[None, :], wall, ball, gfull, pmat,
      Wc1, bc1[None, :], Wc2, bc2[None, :], Wo, bo[None, :])
    return (full_row, jnp.squeeze(score, axis=1))


# speedup vs baseline: 1.1142x; 1.1142x over previous
import jax
import jax.numpy as jnp
import numpy as np
from jax.experimental import pallas as pl

_B = 4096
_ZDIM = 128
_COND = 100
_HID = 256
_N_CONT = 20
_MODES = 10
_N_DISC = 10
_CATS = 10
_PAC = 8
_TAU = 0.2
_ROW = 320
_NG = _N_CONT + _N_DISC
_GRP = _MODES * _NG
_BLK = 512

_HIGH = jax.lax.Precision.HIGHEST


def _perm_matrix() -> np.ndarray:
    dst = np.zeros((_ROW,), np.int64)
    for s in range(_N_CONT):
        dst[s] = 11 * s
    for m in range(_MODES):
        for g in range(_NG):
            s = _N_CONT + _NG * m + g
            if g < _N_CONT:
                dst[s] = 11 * g + 1 + m
            else:
                dst[s] = 220 + 10 * (g - _N_CONT) + m
    P = np.zeros((_ROW, _ROW), np.float32)
    P[np.arange(_ROW), dst] = 1.0
    return P


_PERM_NP = _perm_matrix()


def _gen_body(z_ref, c_ref, w1_ref, b8_ref, w2_ref,
              wall_ref, ball_ref, g_ref, p_ref, out_ref):
    isq = 1.0 / jnp.sqrt(jnp.float32(1.0 + 1e-3))
    h1 = jnp.dot(z_ref[...], w1_ref[0:_ZDIM, :])
    h1 = h1 + jnp.dot(c_ref[...], w1_ref[_ZDIM:_ZDIM + _COND, :])
    h1 = b8_ref[1:2, :] * ((h1 + b8_ref[0:1, :]) * isq) + b8_ref[2:3, :]
    h1 = jnp.maximum(h1, 0.0)
    h2 = jnp.dot(h1, w2_ref[...])
    h2 = b8_ref[4:5, :] * ((h2 + b8_ref[3:4, :]) * isq) + b8_ref[5:6, :]
    h2 = jnp.maximum(h2, 0.0)
    t = jnp.dot(h2, wall_ref[...]) + ball_ref[...]
    a = jnp.tanh(t[:, :_N_CONT])
    s = (t[:, _N_CONT:] + g_ref[...]) / _TAU
    sl = [s[:, k * _NG:(k + 1) * _NG] for k in range(_MODES)]
    m = sl[0]
    for k in range(1, _MODES):
        m = jnp.maximum(m, sl[k])
    es = [jnp.exp(v - m) for v in sl]
    den = es[0]
    for k in range(1, _MODES):
        den = den + es[k]
    ys = [e / den for e in es]
    ymax = ys[0]
    for k in range(1, _MODES):
        ymax = jnp.maximum(ymax, ys[k])
    outs = [((y == ymax).astype(jnp.float32) - y) + y for y in ys]
    u = jnp.concatenate([a] + outs, axis=1)
    out_ref[...] = jnp.dot(u, p_ref[...], precision=_HIGH)


def _critic_body(x_ref, w1_ref, b8_ref, w2_ref, wo_ref, out_ref):
    h = jnp.dot(x_ref[...], w1_ref[...], precision=_HIGH) + b8_ref[6:7, :]
    h = jnp.where(h >= 0, h, 0.2 * h)
    h = jnp.dot(h, w2_ref[...], precision=_HIGH) + b8_ref[7:8, :]
    h = jnp.where(h >= 0, h, 0.2 * h)
    out_ref[...] = jnp.dot(h, wo_ref[...], precision=_HIGH) + b8_ref[8:9, 0:1]


def _gumbel_noise() -> jnp.ndarray:
    gkey = jax.random.key(42)

    def draw(k):
        U = jax.random.uniform(k, (_B, _MODES), minval=1e-6, maxval=1.0 - 1e-6)
        return -jnp.log(-jnp.log(U + 1e-20) + 1e-20)

    kc = jax.vmap(lambda i: jax.random.fold_in(gkey, i))(jnp.arange(_N_CONT))
    kd = jax.vmap(lambda i: jax.random.fold_in(gkey, i))(jnp.arange(1000, 1000 + _N_DISC))
    gc = jax.vmap(draw)(kc)
    gd = jax.vmap(draw)(kd)
    g = jnp.concatenate([jnp.transpose(gc, (1, 2, 0)),
                         jnp.transpose(gd, (1, 2, 0))], axis=2)
    return g.reshape(_B, _GRP)


def kernel(z, cond, W1, b1, g1, be1, W2, b2, g2, be2, Wa, ba, Wb, bb, Wd, bd,
           Wc1, bc1, Wc2, bc2, Wo, bo):
    wa = jnp.transpose(jnp.squeeze(Wa, -1))
    wg = jnp.concatenate([jnp.transpose(Wb, (1, 2, 0)),
                          jnp.transpose(Wd, (1, 2, 0))], axis=2)
    wall = jnp.concatenate([wa, wg.reshape(_HID, _GRP)], axis=1)
    bgrp = jnp.concatenate([jnp.transpose(bb), jnp.transpose(bd)], axis=1).reshape(_GRP)
    ball = jnp.concatenate([jnp.squeeze(ba, -1), bgrp])[None, :]
    gfull = _gumbel_noise()
    pmat = jnp.asarray(_PERM_NP)
    b9 = jnp.stack([b1, g1, be1, b2, g2, be2, bc1, bc2,
                    jnp.concatenate([bo, jnp.zeros((_HID - 1,), jnp.float32)])])

    nblk = _B // _BLK
    full_row = pl.pallas_call(
        _gen_body,
        grid=(nblk,),
        in_specs=[
            pl.BlockSpec((_BLK, _ZDIM), lambda i: (i, 0)),
            pl.BlockSpec((_BLK, _COND), lambda i: (i, 0)),
            pl.BlockSpec((_ZDIM + _COND, _HID), lambda i: (0, 0)),
            pl.BlockSpec((9, _HID), lambda i: (0, 0)),
            pl.BlockSpec((_HID, _HID), lambda i: (0, 0)),
            pl.BlockSpec((_HID, _ROW), lambda i: (0, 0)),
            pl.BlockSpec((1, _ROW), lambda i: (0, 0)),
            pl.BlockSpec((_BLK, _GRP), lambda i: (i, 0)),
            pl.BlockSpec((_ROW, _ROW), lambda i: (0, 0)),
        ],
        out_specs=pl.BlockSpec((_BLK, _ROW), lambda i: (i, 0)),
        out_shape=jax.ShapeDtypeStruct((_B, _ROW), jnp.float32),
    )(z, cond, W1, b9, W2, wall, ball, gfull, pmat)

    xr = full_row.reshape(_B // _PAC, _PAC * _ROW)
    score = pl.pallas_call(
        _critic_body,
        out_shape=jax.ShapeDtypeStruct((_B // _PAC, 1), jnp.float32),
    )(xr, Wc1, b9, Wc2, Wo)
    return (full_row, jnp.squeeze(score, axis=1))


# speedup vs baseline: 1.5653x; 1.4049x over previous
import jax
import jax.numpy as jnp
import numpy as np
from jax.experimental import pallas as pl

_B = 4096
_ZDIM = 128
_COND = 100
_HID = 256
_N_CONT = 20
_MODES = 10
_N_DISC = 10
_CATS = 10
_PAC = 8
_TAU = 0.2
_ROW = 320
_NG = _N_CONT + _N_DISC
_GRP = _MODES * _NG
_BLK = 1024

_HIGH = jax.lax.Precision.HIGHEST


def _perm_matrix() -> np.ndarray:
    dst = np.zeros((_ROW,), np.int64)
    for s in range(_N_CONT):
        dst[s] = 11 * s
    for m in range(_MODES):
        for g in range(_NG):
            s = _N_CONT + _NG * m + g
            if g < _N_CONT:
                dst[s] = 11 * g + 1 + m
            else:
                dst[s] = 220 + 10 * (g - _N_CONT) + m
    P = np.zeros((_ROW, _ROW), np.float32)
    P[np.arange(_ROW), dst] = 1.0
    return P


_PERM_NP = _perm_matrix()


def _gen_body(z_ref, c_ref, w1_ref, b8_ref, w2_ref,
              wall_ref, ball_ref, g_ref, p_ref, out_ref):
    isq = 1.0 / jnp.sqrt(jnp.float32(1.0 + 1e-3))
    h1 = jnp.dot(z_ref[...], w1_ref[0:_ZDIM, :])
    h1 = h1 + jnp.dot(c_ref[...], w1_ref[_ZDIM:_ZDIM + _COND, :])
    h1 = b8_ref[1:2, :] * ((h1 + b8_ref[0:1, :]) * isq) + b8_ref[2:3, :]
    h1 = jnp.maximum(h1, 0.0)
    h2 = jnp.dot(h1, w2_ref[...])
    h2 = b8_ref[4:5, :] * ((h2 + b8_ref[3:4, :]) * isq) + b8_ref[5:6, :]
    h2 = jnp.maximum(h2, 0.0)
    t = jnp.dot(h2, wall_ref[...]) + ball_ref[...]
    a = jnp.tanh(t[:, :_N_CONT])
    s = (t[:, _N_CONT:] + g_ref[...]) / _TAU
    sl = [s[:, k * _NG:(k + 1) * _NG] for k in range(_MODES)]
    m = sl[0]
    for k in range(1, _MODES):
        m = jnp.maximum(m, sl[k])
    es = [jnp.exp(v - m) for v in sl]
    den = es[0]
    for k in range(1, _MODES):
        den = den + es[k]
    ys = [e / den for e in es]
    ymax = ys[0]
    for k in range(1, _MODES):
        ymax = jnp.maximum(ymax, ys[k])
    outs = [((y == ymax).astype(jnp.float32) - y) + y for y in ys]
    u = jnp.concatenate([a] + outs, axis=1)
    out_ref[...] = jnp.dot(u, p_ref[...], precision=_HIGH)


def _critic_body(x_ref, w1_ref, b8_ref, w2_ref, wo_ref, out_ref):
    h = jnp.dot(x_ref[...], w1_ref[...], precision=_HIGH) + b8_ref[6:7, :]
    h = jnp.where(h >= 0, h, 0.2 * h)
    h = jnp.dot(h, w2_ref[...], precision=_HIGH) + b8_ref[7:8, :]
    h = jnp.where(h >= 0, h, 0.2 * h)
    out_ref[...] = jnp.dot(h, wo_ref[...], precision=_HIGH) + b8_ref[8:9, 0:1]


def _gumbel_noise() -> jnp.ndarray:
    with jax.ensure_compile_time_eval():
        return _gumbel_noise_impl()


def _gumbel_noise_impl() -> jnp.ndarray:
    gkey = jax.random.key(42)

    def draw(k):
        U = jax.random.uniform(k, (_B, _MODES), minval=1e-6, maxval=1.0 - 1e-6)
        return -jnp.log(-jnp.log(U + 1e-20) + 1e-20)

    kc = jax.vmap(lambda i: jax.random.fold_in(gkey, i))(jnp.arange(_N_CONT))
    kd = jax.vmap(lambda i: jax.random.fold_in(gkey, i))(jnp.arange(1000, 1000 + _N_DISC))
    gc = jax.vmap(draw)(kc)
    gd = jax.vmap(draw)(kd)
    g = jnp.concatenate([jnp.transpose(gc, (1, 2, 0)),
                         jnp.transpose(gd, (1, 2, 0))], axis=2)
    return g.reshape(_B, _GRP)


def kernel(z, cond, W1, b1, g1, be1, W2, b2, g2, be2, Wa, ba, Wb, bb, Wd, bd,
           Wc1, bc1, Wc2, bc2, Wo, bo):
    wa = jnp.transpose(jnp.squeeze(Wa, -1))
    wg = jnp.concatenate([jnp.transpose(Wb, (1, 2, 0)),
                          jnp.transpose(Wd, (1, 2, 0))], axis=2)
    wall = jnp.concatenate([wa, wg.reshape(_HID, _GRP)], axis=1)
    bgrp = jnp.concatenate([jnp.transpose(bb), jnp.transpose(bd)], axis=1).reshape(_GRP)
    ball = jnp.concatenate([jnp.squeeze(ba, -1), bgrp])[None, :]
    gfull = _gumbel_noise()
    pmat = jnp.asarray(_PERM_NP)
    b9 = jnp.stack([b1, g1, be1, b2, g2, be2, bc1, bc2,
                    jnp.concatenate([bo, jnp.zeros((_HID - 1,), jnp.float32)])])

    nblk = _B // _BLK
    full_row = pl.pallas_call(
        _gen_body,
        grid=(nblk,),
        in_specs=[
            pl.BlockSpec((_BLK, _ZDIM), lambda i: (i, 0)),
            pl.BlockSpec((_BLK, _COND), lambda i: (i, 0)),
            pl.BlockSpec((_ZDIM + _COND, _HID), lambda i: (0, 0)),
            pl.BlockSpec((9, _HID), lambda i: (0, 0)),
            pl.BlockSpec((_HID, _HID), lambda i: (0, 0)),
            pl.BlockSpec((_HID, _ROW), lambda i: (0, 0)),
            pl.BlockSpec((1, _ROW), lambda i: (0, 0)),
            pl.BlockSpec((_BLK, _GRP), lambda i: (i, 0)),
            pl.BlockSpec((_ROW, _ROW), lambda i: (0, 0)),
        ],
        out_specs=pl.BlockSpec((_BLK, _ROW), lambda i: (i, 0)),
        out_shape=jax.ShapeDtypeStruct((_B, _ROW), jnp.float32),
    )(z, cond, W1, b9, W2, wall, ball, gfull, pmat)

    xr = full_row.reshape(_B // _PAC, _PAC * _ROW)
    score = pl.pallas_call(
        _critic_body,
        out_shape=jax.ShapeDtypeStruct((_B // _PAC, 1), jnp.float32),
    )(xr, Wc1, b9, Wc2, Wo)
    return (full_row, jnp.squeeze(score, axis=1))


# speedup vs baseline: 1.6682x; 1.0657x over previous
import jax
import jax.numpy as jnp
import numpy as np
from jax.experimental import pallas as pl

_B = 4096
_ZDIM = 128
_COND = 100
_HID = 256
_N_CONT = 20
_MODES = 10
_N_DISC = 10
_CATS = 10
_PAC = 8
_TAU = 0.2
_ROW = 320
_NG = _N_CONT + _N_DISC
_GRP = _MODES * _NG
_BLK = 1024

_HIGH = jax.lax.Precision.HIGHEST
_H3 = jax.lax.Precision.HIGH


def _perm_matrix() -> np.ndarray:
    dst = np.zeros((_ROW,), np.int64)
    for s in range(_N_CONT):
        dst[s] = 11 * s
    for m in range(_MODES):
        for g in range(_NG):
            s = _N_CONT + _NG * m + g
            if g < _N_CONT:
                dst[s] = 11 * g + 1 + m
            else:
                dst[s] = 220 + 10 * (g - _N_CONT) + m
    P = np.zeros((_ROW, _ROW), np.float32)
    P[np.arange(_ROW), dst] = 1.0
    return P


_PERM_NP = _perm_matrix()


def _gen_body(z_ref, c_ref, w1_ref, b8_ref, w2_ref,
              wall_ref, ball_ref, g_ref, p_ref, out_ref):
    isq = 1.0 / jnp.sqrt(jnp.float32(1.0 + 1e-3))
    h1 = jnp.dot(z_ref[...], w1_ref[0:_ZDIM, :])
    h1 = h1 + jnp.dot(c_ref[...], w1_ref[_ZDIM:_ZDIM + _COND, :])
    h1 = b8_ref[1:2, :] * ((h1 + b8_ref[0:1, :]) * isq) + b8_ref[2:3, :]
    h1 = jnp.maximum(h1, 0.0)
    h2 = jnp.dot(h1, w2_ref[...])
    h2 = b8_ref[4:5, :] * ((h2 + b8_ref[3:4, :]) * isq) + b8_ref[5:6, :]
    h2 = jnp.maximum(h2, 0.0)
    t = jnp.dot(h2, wall_ref[...]) + ball_ref[...]
    a = jnp.tanh(t[:, :_N_CONT])
    s = (t[:, _N_CONT:] + g_ref[...]) / _TAU
    sl = [s[:, k * _NG:(k + 1) * _NG] for k in range(_MODES)]
    m = sl[0]
    for k in range(1, _MODES):
        m = jnp.maximum(m, sl[k])
    es = [jnp.exp(v - m) for v in sl]
    den = es[0]
    for k in range(1, _MODES):
        den = den + es[k]
    ys = [e / den for e in es]
    ymax = ys[0]
    for k in range(1, _MODES):
        ymax = jnp.maximum(ymax, ys[k])
    outs = [((y == ymax).astype(jnp.float32) - y) + y for y in ys]
    u = jnp.concatenate([a] + outs, axis=1)
    out_ref[...] = jnp.dot(u, p_ref[...], precision=_HIGH)


def _critic_body(x_ref, w1_ref, b8_ref, w2_ref, wo_ref, out_ref):
    h = jnp.dot(x_ref[...], w1_ref[...]) + b8_ref[6:7, :]
    h = jnp.where(h >= 0, h, 0.2 * h)
    h = jnp.dot(h, w2_ref[...]) + b8_ref[7:8, :]
    h = jnp.where(h >= 0, h, 0.2 * h)
    out_ref[...] = jnp.dot(h, wo_ref[...]) + b8_ref[8:9, 0:1]


def _gumbel_noise() -> jnp.ndarray:
    try:
        with jax.ensure_compile_time_eval():
            return _gumbel_noise_impl()
    except Exception:
        return _gumbel_noise_impl()


def _gumbel_noise_impl() -> jnp.ndarray:
    gkey = jax.random.key(42)

    def draw(k):
        U = jax.random.uniform(k, (_B, _MODES), minval=1e-6, maxval=1.0 - 1e-6)
        return -jnp.log(-jnp.log(U + 1e-20) + 1e-20)

    kc = jax.vmap(lambda i: jax.random.fold_in(gkey, i))(jnp.arange(_N_CONT))
    kd = jax.vmap(lambda i: jax.random.fold_in(gkey, i))(jnp.arange(1000, 1000 + _N_DISC))
    gc = jax.vmap(draw)(kc)
    gd = jax.vmap(draw)(kd)
    g = jnp.concatenate([jnp.transpose(gc, (1, 2, 0)),
                         jnp.transpose(gd, (1, 2, 0))], axis=2)
    return g.reshape(_B, _GRP)


def kernel(z, cond, W1, b1, g1, be1, W2, b2, g2, be2, Wa, ba, Wb, bb, Wd, bd,
           Wc1, bc1, Wc2, bc2, Wo, bo):
    wa = jnp.transpose(jnp.squeeze(Wa, -1))
    wg = jnp.concatenate([jnp.transpose(Wb, (1, 2, 0)),
                          jnp.transpose(Wd, (1, 2, 0))], axis=2)
    wall = jnp.concatenate([wa, wg.reshape(_HID, _GRP)], axis=1)
    bgrp = jnp.concatenate([jnp.transpose(bb), jnp.transpose(bd)], axis=1).reshape(_GRP)
    ball = jnp.concatenate([jnp.squeeze(ba, -1), bgrp])[None, :]
    gfull = _gumbel_noise()
    pmat = jnp.asarray(_PERM_NP)
    b9 = jnp.stack([b1, g1, be1, b2, g2, be2, bc1, bc2,
                    jnp.concatenate([bo, jnp.zeros((_HID - 1,), jnp.float32)])])

    nblk = _B // _BLK
    full_row = pl.pallas_call(
        _gen_body,
        grid=(nblk,),
        in_specs=[
            pl.BlockSpec((_BLK, _ZDIM), lambda i: (i, 0)),
            pl.BlockSpec((_BLK, _COND), lambda i: (i, 0)),
            pl.BlockSpec((_ZDIM + _COND, _HID), lambda i: (0, 0)),
            pl.BlockSpec((9, _HID), lambda i: (0, 0)),
            pl.BlockSpec((_HID, _HID), lambda i: (0, 0)),
            pl.BlockSpec((_HID, _ROW), lambda i: (0, 0)),
            pl.BlockSpec((1, _ROW), lambda i: (0, 0)),
            pl.BlockSpec((_BLK, _GRP), lambda i: (i, 0)),
            pl.BlockSpec((_ROW, _ROW), lambda i: (0, 0)),
        ],
        out_specs=pl.BlockSpec((_BLK, _ROW), lambda i: (i, 0)),
        out_shape=jax.ShapeDtypeStruct((_B, _ROW), jnp.float32),
    )(z, cond, W1, b9, W2, wall, ball, gfull, pmat)

    xr = full_row.reshape(_B // _PAC, _PAC * _ROW)
    score = pl.pallas_call(
        _critic_body,
        out_shape=jax.ShapeDtypeStruct((_B // _PAC, 1), jnp.float32),
    )(xr, Wc1, b9, Wc2, Wo)
    return (full_row, jnp.squeeze(score, axis=1))


# speedup vs baseline: 1.8891x; 1.1324x over previous
import jax
import jax.numpy as jnp
import numpy as np
from jax.experimental import pallas as pl

_B = 4096
_ZDIM = 128
_COND = 100
_HID = 256
_N_CONT = 20
_MODES = 10
_N_DISC = 10
_CATS = 10
_PAC = 8
_TAU = 0.2
_ROW = 320
_NG = _N_CONT + _N_DISC
_GRP = _MODES * _NG
_BLK = 1024

_HIGH = jax.lax.Precision.HIGHEST
_H3 = jax.lax.Precision.HIGH


def _perm_matrix() -> np.ndarray:
    dst = np.zeros((_ROW,), np.int64)
    for s in range(_N_CONT):
        dst[s] = 11 * s
    for m in range(_MODES):
        for g in range(_NG):
            s = _N_CONT + _NG * m + g
            if g < _N_CONT:
                dst[s] = 11 * g + 1 + m
            else:
                dst[s] = 220 + 10 * (g - _N_CONT) + m
    P = np.zeros((_ROW, _ROW), np.float32)
    P[np.arange(_ROW), dst] = 1.0
    return P


_PERM_NP = _perm_matrix()


def _gen_body(z_ref, c_ref, w1_ref, b8_ref, w2_ref,
              wall_ref, ball_ref, g_ref, p_ref, out_ref):
    isq = 1.0 / jnp.sqrt(jnp.float32(1.0 + 1e-3))
    h1 = jnp.dot(z_ref[...], w1_ref[0:_ZDIM, :])
    h1 = h1 + jnp.dot(c_ref[...], w1_ref[_ZDIM:_ZDIM + _COND, :])
    h1 = b8_ref[1:2, :] * ((h1 + b8_ref[0:1, :]) * isq) + b8_ref[2:3, :]
    h1 = jnp.maximum(h1, 0.0)
    h2 = jnp.dot(h1, w2_ref[...])
    h2 = b8_ref[4:5, :] * ((h2 + b8_ref[3:4, :]) * isq) + b8_ref[5:6, :]
    h2 = jnp.maximum(h2, 0.0)
    t = jnp.dot(h2, wall_ref[...]) + ball_ref[...]
    a = jnp.tanh(t[:, :_N_CONT])
    s = (t[:, _N_CONT:] + g_ref[...]) / _TAU
    sl = [s[:, k * _NG:(k + 1) * _NG] for k in range(_MODES)]
    m = sl[0]
    for k in range(1, _MODES):
        m = jnp.maximum(m, sl[k])
    es = [jnp.exp(v - m) for v in sl]
    den = es[0]
    for k in range(1, _MODES):
        den = den + es[k]
    ys = [e / den for e in es]
    ymax = ys[0]
    for k in range(1, _MODES):
        ymax = jnp.maximum(ymax, ys[k])
    outs = [((y == ymax).astype(jnp.float32) - y) + y for y in ys]
    u = jnp.concatenate([a] + outs, axis=1)
    out_ref[...] = jnp.dot(u, p_ref[...])


def _critic_body(x_ref, w1_ref, b8_ref, w2_ref, wo_ref, out_ref):
    h = jnp.dot(x_ref[...], w1_ref[...]) + b8_ref[6:7, :]
    h = jnp.where(h >= 0, h, 0.2 * h)
    h = jnp.dot(h, w2_ref[...]) + b8_ref[7:8, :]
    h = jnp.where(h >= 0, h, 0.2 * h)
    out_ref[...] = jnp.dot(h, wo_ref[...]) + b8_ref[8:9, 0:1]


def _gumbel_noise() -> jnp.ndarray:
    try:
        with jax.ensure_compile_time_eval():
            return _gumbel_noise_impl()
    except Exception:
        return _gumbel_noise_impl()


def _gumbel_noise_impl() -> jnp.ndarray:
    gkey = jax.random.key(42)

    def draw(k):
        U = jax.random.uniform(k, (_B, _MODES), minval=1e-6, maxval=1.0 - 1e-6)
        return -jnp.log(-jnp.log(U + 1e-20) + 1e-20)

    kc = jax.vmap(lambda i: jax.random.fold_in(gkey, i))(jnp.arange(_N_CONT))
    kd = jax.vmap(lambda i: jax.random.fold_in(gkey, i))(jnp.arange(1000, 1000 + _N_DISC))
    gc = jax.vmap(draw)(kc)
    gd = jax.vmap(draw)(kd)
    g = jnp.concatenate([jnp.transpose(gc, (1, 2, 0)),
                         jnp.transpose(gd, (1, 2, 0))], axis=2)
    return g.reshape(_B, _GRP)


def kernel(z, cond, W1, b1, g1, be1, W2, b2, g2, be2, Wa, ba, Wb, bb, Wd, bd,
           Wc1, bc1, Wc2, bc2, Wo, bo):
    wa = jnp.transpose(jnp.squeeze(Wa, -1))
    wg = jnp.concatenate([jnp.transpose(Wb, (1, 2, 0)),
                          jnp.transpose(Wd, (1, 2, 0))], axis=2)
    wall = jnp.concatenate([wa, wg.reshape(_HID, _GRP)], axis=1)
    bgrp = jnp.concatenate([jnp.transpose(bb), jnp.transpose(bd)], axis=1).reshape(_GRP)
    ball = jnp.concatenate([jnp.squeeze(ba, -1), bgrp])[None, :]
    gfull = _gumbel_noise()
    pmat = jnp.asarray(_PERM_NP)
    b9 = jnp.stack([b1, g1, be1, b2, g2, be2, bc1, bc2,
                    jnp.concatenate([bo, jnp.zeros((_HID - 1,), jnp.float32)])])

    nblk = _B // _BLK
    full_row = pl.pallas_call(
        _gen_body,
        grid=(nblk,),
        in_specs=[
            pl.BlockSpec((_BLK, _ZDIM), lambda i: (i, 0)),
            pl.BlockSpec((_BLK, _COND), lambda i: (i, 0)),
            pl.BlockSpec((_ZDIM + _COND, _HID), lambda i: (0, 0)),
            pl.BlockSpec((9, _HID), lambda i: (0, 0)),
            pl.BlockSpec((_HID, _HID), lambda i: (0, 0)),
            pl.BlockSpec((_HID, _ROW), lambda i: (0, 0)),
            pl.BlockSpec((1, _ROW), lambda i: (0, 0)),
            pl.BlockSpec((_BLK, _GRP), lambda i: (i, 0)),
            pl.BlockSpec((_ROW, _ROW), lambda i: (0, 0)),
        ],
        out_specs=pl.BlockSpec((_BLK, _ROW), lambda i: (i, 0)),
        out_shape=jax.ShapeDtypeStruct((_B, _ROW), jnp.float32),
    )(z, cond, W1, b9, W2, wall, ball, gfull, pmat)

    xr = full_row.reshape(_B // _PAC, _PAC * _ROW)
    score = pl.pallas_call(
        _critic_body,
        out_shape=jax.ShapeDtypeStruct((_B // _PAC, 1), jnp.float32),
    )(xr, Wc1, b9, Wc2, Wo)
    return (full_row, jnp.squeeze(score, axis=1))


# speedup vs baseline: 2.2689x; 1.2010x over previous
import jax
import jax.numpy as jnp
import numpy as np
from jax.experimental import pallas as pl

_B = 4096
_ZDIM = 128
_COND = 100
_HID = 256
_N_CONT = 20
_MODES = 10
_N_DISC = 10
_CATS = 10
_PAC = 8
_TAU = 0.2
_ROW = 320
_NG = _N_CONT + _N_DISC
_NGP = 32
_GRP = _MODES * _NGP
_WID = _GRP + _N_CONT
_BLK = 1024

_HIGH = jax.lax.Precision.HIGHEST
_H3 = jax.lax.Precision.HIGH


def _perm_matrix() -> np.ndarray:
    P = np.zeros((_WID, _ROW), np.float32)
    for m in range(_MODES):
        for g in range(_NG):
            s = _NGP * m + g
            if g < _N_CONT:
                P[s, 11 * g + 1 + m] = 1.0
            else:
                P[s, 220 + 10 * (g - _N_CONT) + m] = 1.0
    for s in range(_N_CONT):
        P[_GRP + s, 11 * s] = 1.0
    return P


_PERM_NP = _perm_matrix()


def _gen_body(z_ref, c_ref, w1_ref, b8_ref, w2_ref,
              wall_ref, ball_ref, g_ref, p_ref, out_ref):
    isq = 1.0 / jnp.sqrt(jnp.float32(1.0 + 1e-3))
    h1 = jnp.dot(z_ref[...], w1_ref[0:_ZDIM, :])
    h1 = h1 + jnp.dot(c_ref[...], w1_ref[_ZDIM:_ZDIM + _COND, :])
    h1 = b8_ref[1:2, :] * ((h1 + b8_ref[0:1, :]) * isq) + b8_ref[2:3, :]
    h1 = jnp.maximum(h1, 0.0)
    h2 = jnp.dot(h1, w2_ref[...])
    h2 = b8_ref[4:5, :] * ((h2 + b8_ref[3:4, :]) * isq) + b8_ref[5:6, :]
    h2 = jnp.maximum(h2, 0.0)
    t = jnp.dot(h2, wall_ref[...]) + ball_ref[...]
    a = jnp.tanh(t[:, _GRP:_WID])
    s = (t[:, :_GRP] + g_ref[...]) / _TAU
    sl = [s[:, k * _NGP:(k + 1) * _NGP] for k in range(_MODES)]
    m = sl[0]
    for k in range(1, _MODES):
        m = jnp.maximum(m, sl[k])
    es = [jnp.exp(v - m) for v in sl]
    den = es[0]
    for k in range(1, _MODES):
        den = den + es[k]
    ys = [e / den for e in es]
    ymax = ys[0]
    for k in range(1, _MODES):
        ymax = jnp.maximum(ymax, ys[k])
    outs = [((y == ymax).astype(jnp.float32) - y) + y for y in ys]
    u = jnp.concatenate(outs + [a], axis=1)
    out_ref[...] = jnp.dot(u, p_ref[...])


def _critic_body(x_ref, w1_ref, b8_ref, w2_ref, wo_ref, out_ref):
    h = jnp.dot(x_ref[...], w1_ref[...]) + b8_ref[6:7, :]
    h = jnp.where(h >= 0, h, 0.2 * h)
    h = jnp.dot(h, w2_ref[...]) + b8_ref[7:8, :]
    h = jnp.where(h >= 0, h, 0.2 * h)
    out_ref[...] = jnp.dot(h, wo_ref[...]) + b8_ref[8:9, 0:1]


def _gumbel_noise() -> jnp.ndarray:
    try:
        with jax.ensure_compile_time_eval():
            return _gumbel_noise_impl()
    except Exception:
        return _gumbel_noise_impl()


def _gumbel_noise_impl() -> jnp.ndarray:
    gkey = jax.random.key(42)

    def draw(k):
        U = jax.random.uniform(k, (_B, _MODES), minval=1e-6, maxval=1.0 - 1e-6)
        return -jnp.log(-jnp.log(U + 1e-20) + 1e-20)

    kc = jax.vmap(lambda i: jax.random.fold_in(gkey, i))(jnp.arange(_N_CONT))
    kd = jax.vmap(lambda i: jax.random.fold_in(gkey, i))(jnp.arange(1000, 1000 + _N_DISC))
    gc = jax.vmap(draw)(kc)
    gd = jax.vmap(draw)(kd)
    g = jnp.concatenate([jnp.transpose(gc, (1, 2, 0)),
                         jnp.transpose(gd, (1, 2, 0)),
                         jnp.zeros((_B, _MODES, _NGP - _NG), jnp.float32)],
                        axis=2)
    return g.reshape(_B, _GRP)


def kernel(z, cond, W1, b1, g1, be1, W2, b2, g2, be2, Wa, ba, Wb, bb, Wd, bd,
           Wc1, bc1, Wc2, bc2, Wo, bo):
    wa = jnp.transpose(jnp.squeeze(Wa, -1))
    wg = jnp.concatenate([jnp.transpose(Wb, (1, 2, 0)),
                          jnp.transpose(Wd, (1, 2, 0)),
                          jnp.zeros((_HID, _MODES, _NGP - _NG), jnp.float32)],
                         axis=2)
    wall = jnp.concatenate([wg.reshape(_HID, _GRP), wa], axis=1)
    bgrp = jnp.concatenate([jnp.transpose(bb), jnp.transpose(bd),
                            jnp.zeros((_MODES, _NGP - _NG), jnp.float32)],
                           axis=1).reshape(_GRP)
    ball = jnp.concatenate([bgrp, jnp.squeeze(ba, -1)])[None, :]
    gfull = _gumbel_noise()
    pmat = jnp.asarray(_PERM_NP)
    b9 = jnp.stack([b1, g1, be1, b2, g2, be2, bc1, bc2,
                    jnp.concatenate([bo, jnp.zeros((_HID - 1,), jnp.float32)])])

    nblk = _B // _BLK
    full_row = pl.pallas_call(
        _gen_body,
        grid=(nblk,),
        in_specs=[
            pl.BlockSpec((_BLK, _ZDIM), lambda i: (i, 0)),
            pl.BlockSpec((_BLK, _COND), lambda i: (i, 0)),
            pl.BlockSpec((_ZDIM + _COND, _HID), lambda i: (0, 0)),
            pl.BlockSpec((9, _HID), lambda i: (0, 0)),
            pl.BlockSpec((_HID, _HID), lambda i: (0, 0)),
            pl.BlockSpec((_HID, _WID), lambda i: (0, 0)),
            pl.BlockSpec((1, _WID), lambda i: (0, 0)),
            pl.BlockSpec((_BLK, _GRP), lambda i: (i, 0)),
            pl.BlockSpec((_WID, _ROW), lambda i: (0, 0)),
        ],
        out_specs=pl.BlockSpec((_BLK, _ROW), lambda i: (i, 0)),
        out_shape=jax.ShapeDtypeStruct((_B, _ROW), jnp.float32),
    )(z, cond, W1, b9, W2, wall, ball, gfull, pmat)

    xr = full_row.reshape(_B // _PAC, _PAC * _ROW)
    score = pl.pallas_call(
        _critic_body,
        out_shape=jax.ShapeDtypeStruct((_B // _PAC, 1), jnp.float32),
    )(xr, Wc1, b9, Wc2, Wo)
    return (full_row, jnp.squeeze(score, axis=1))


# speedup vs baseline: 2.7393x; 1.2073x over previous
import jax
import jax.numpy as jnp
import numpy as np
from jax.experimental import pallas as pl

_B = 4096
_ZDIM = 128
_COND = 100
_HID = 256
_N_CONT = 20
_MODES = 10
_N_DISC = 10
_CATS = 10
_PAC = 8
_TAU = 0.2
_ROW = 320
_NG = _N_CONT + _N_DISC
_NGP = 32
_GRP = _MODES * _NGP
_WID = _GRP + _N_CONT
_BLK = 1024

_HIGH = jax.lax.Precision.HIGHEST
_H3 = jax.lax.Precision.HIGH


def _perm_matrix() -> np.ndarray:
    P = np.zeros((_WID, _ROW), np.float32)
    for m in range(_MODES):
        for g in range(_NG):
            s = _NGP * m + g
            if g < _N_CONT:
                P[s, 11 * g + 1 + m] = 1.0
            else:
                P[s, 220 + 10 * (g - _N_CONT) + m] = 1.0
    for s in range(_N_CONT):
        P[_GRP + s, 11 * s] = 1.0
    return P


_PERM_NP = _perm_matrix()


def _gen_body(z_ref, c_ref, w1_ref, b8_ref, w2_ref,
              wall_ref, ball_ref, g_ref, p_ref,
              wc1_ref, wc2_ref, wo_ref, out_ref, sc_ref):
    isq = 1.0 / jnp.sqrt(jnp.float32(1.0 + 1e-3))
    h1 = jnp.dot(z_ref[...], w1_ref[0:_ZDIM, :])
    h1 = h1 + jnp.dot(c_ref[...], w1_ref[_ZDIM:_ZDIM + _COND, :])
    h1 = b8_ref[1:2, :] * ((h1 + b8_ref[0:1, :]) * isq) + b8_ref[2:3, :]
    h1 = jnp.maximum(h1, 0.0)
    h2 = jnp.dot(h1, w2_ref[...])
    h2 = b8_ref[4:5, :] * ((h2 + b8_ref[3:4, :]) * isq) + b8_ref[5:6, :]
    h2 = jnp.maximum(h2, 0.0)
    t = jnp.dot(h2, wall_ref[...]) + ball_ref[...]
    a = jnp.tanh(t[:, _GRP:_WID])
    s = (t[:, :_GRP] + g_ref[...]) / _TAU
    sl = [s[:, k * _NGP:(k + 1) * _NGP] for k in range(_MODES)]
    m = sl[0]
    for k in range(1, _MODES):
        m = jnp.maximum(m, sl[k])
    es = [jnp.exp(v - m) for v in sl]
    den = es[0]
    for k in range(1, _MODES):
        den = den + es[k]
    ys = [e / den for e in es]
    ymax = ys[0]
    for k in range(1, _MODES):
        ymax = jnp.maximum(ymax, ys[k])
    outs = [((y == ymax).astype(jnp.float32) - y) + y for y in ys]
    u = jnp.concatenate(outs + [a], axis=1)
    fr = jnp.dot(u, p_ref[...])
    out_ref[...] = fr
    fr3 = fr.reshape(_BLK // _PAC, _PAC, _ROW)
    hc = jnp.dot(fr3[:, 0, :], wc1_ref[0:_ROW, :])
    for p in range(1, _PAC):
        hc = hc + jnp.dot(fr3[:, p, :], wc1_ref[p * _ROW:(p + 1) * _ROW, :])
    hc = hc + b8_ref[6:7, :]
    hc = jnp.where(hc >= 0, hc, 0.2 * hc)
    hc = jnp.dot(hc, wc2_ref[...]) + b8_ref[7:8, :]
    hc = jnp.where(hc >= 0, hc, 0.2 * hc)
    sc_ref[...] = jnp.dot(hc, wo_ref[...]) + b8_ref[8:9, 0:1]


def _critic_body(x_ref, w1_ref, b8_ref, w2_ref, wo_ref, out_ref):
    h = jnp.dot(x_ref[...], w1_ref[...]) + b8_ref[6:7, :]
    h = jnp.where(h >= 0, h, 0.2 * h)
    h = jnp.dot(h, w2_ref[...]) + b8_ref[7:8, :]
    h = jnp.where(h >= 0, h, 0.2 * h)
    out_ref[...] = jnp.dot(h, wo_ref[...]) + b8_ref[8:9, 0:1]


def _gumbel_noise() -> jnp.ndarray:
    try:
        with jax.ensure_compile_time_eval():
            return _gumbel_noise_impl()
    except Exception:
        return _gumbel_noise_impl()


def _gumbel_noise_impl() -> jnp.ndarray:
    gkey = jax.random.key(42)

    def draw(k):
        U = jax.random.uniform(k, (_B, _MODES), minval=1e-6, maxval=1.0 - 1e-6)
        return -jnp.log(-jnp.log(U + 1e-20) + 1e-20)

    kc = jax.vmap(lambda i: jax.random.fold_in(gkey, i))(jnp.arange(_N_CONT))
    kd = jax.vmap(lambda i: jax.random.fold_in(gkey, i))(jnp.arange(1000, 1000 + _N_DISC))
    gc = jax.vmap(draw)(kc)
    gd = jax.vmap(draw)(kd)
    g = jnp.concatenate([jnp.transpose(gc, (1, 2, 0)),
                         jnp.transpose(gd, (1, 2, 0)),
                         jnp.zeros((_B, _MODES, _NGP - _NG), jnp.float32)],
                        axis=2)
    return g.reshape(_B, _GRP)


def kernel(z, cond, W1, b1, g1, be1, W2, b2, g2, be2, Wa, ba, Wb, bb, Wd, bd,
           Wc1, bc1, Wc2, bc2, Wo, bo):
    wa = jnp.transpose(jnp.squeeze(Wa, -1))
    wg = jnp.concatenate([jnp.transpose(Wb, (1, 2, 0)),
                          jnp.transpose(Wd, (1, 2, 0)),
                          jnp.zeros((_HID, _MODES, _NGP - _NG), jnp.float32)],
                         axis=2)
    wall = jnp.concatenate([wg.reshape(_HID, _GRP), wa], axis=1)
    bgrp = jnp.concatenate([jnp.transpose(bb), jnp.transpose(bd),
                            jnp.zeros((_MODES, _NGP - _NG), jnp.float32)],
                           axis=1).reshape(_GRP)
    ball = jnp.concatenate([bgrp, jnp.squeeze(ba, -1)])[None, :]
    gfull = _gumbel_noise()
    pmat = jnp.asarray(_PERM_NP)
    b9 = jnp.stack([b1, g1, be1, b2, g2, be2, bc1, bc2,
                    jnp.concatenate([bo, jnp.zeros((_HID - 1,), jnp.float32)])])

    nblk = _B // _BLK
    full_row, score = pl.pallas_call(
        _gen_body,
        grid=(nblk,),
        in_specs=[
            pl.BlockSpec((_BLK, _ZDIM), lambda i: (i, 0)),
            pl.BlockSpec((_BLK, _COND), lambda i: (i, 0)),
            pl.BlockSpec((_ZDIM + _COND, _HID), lambda i: (0, 0)),
            pl.BlockSpec((9, _HID), lambda i: (0, 0)),
            pl.BlockSpec((_HID, _HID), lambda i: (0, 0)),
            pl.BlockSpec((_HID, _WID), lambda i: (0, 0)),
            pl.BlockSpec((1, _WID), lambda i: (0, 0)),
            pl.BlockSpec((_BLK, _GRP), lambda i: (i, 0)),
            pl.BlockSpec((_WID, _ROW), lambda i: (0, 0)),
            pl.BlockSpec((_PAC * _ROW, _HID), lambda i: (0, 0)),
            pl.BlockSpec((_HID, _HID), lambda i: (0, 0)),
            pl.BlockSpec((_HID, 1), lambda i: (0, 0)),
        ],
        out_specs=[
            pl.BlockSpec((_BLK, _ROW), lambda i: (i, 0)),
            pl.BlockSpec((_BLK // _PAC, 1), lambda i: (i, 0)),
        ],
        out_shape=[
            jax.ShapeDtypeStruct((_B, _ROW), jnp.float32),
            jax.ShapeDtypeStruct((_B // _PAC, 1), jnp.float32),
        ],
    )(z, cond, W1, b9, W2, wall, ball, gfull, pmat, Wc1, Wc2, Wo)
    return (full_row, jnp.squeeze(score, axis=1))


# speedup vs baseline: 2.7717x; 1.0118x over previous
import jax
import jax.numpy as jnp
import numpy as np
from jax.experimental import pallas as pl

_B = 4096
_ZDIM = 128
_COND = 100
_HID = 256
_N_CONT = 20
_MODES = 10
_N_DISC = 10
_CATS = 10
_PAC = 8
_TAU = 0.2
_ROW = 320
_NG = _N_CONT + _N_DISC
_NGP = 32
_GRP = _MODES * _NGP
_WID = _GRP + _N_CONT
_BLK = 2048

_HIGH = jax.lax.Precision.HIGHEST
_H3 = jax.lax.Precision.HIGH


def _perm_matrix() -> np.ndarray:
    P = np.zeros((_WID, _ROW), np.float32)
    for m in range(_MODES):
        for g in range(_NG):
            s = _NGP * m + g
            if g < _N_CONT:
                P[s, 11 * g + 1 + m] = 1.0
            else:
                P[s, 220 + 10 * (g - _N_CONT) + m] = 1.0
    for s in range(_N_CONT):
        P[_GRP + s, 11 * s] = 1.0
    return P


_PERM_NP = _perm_matrix()


def _gen_body(z_ref, c_ref, w1_ref, b8_ref, w2_ref,
              wall_ref, ball_ref, g_ref, p_ref,
              wc1_ref, wc2_ref, wo_ref, out_ref, sc_ref):
    isq = 1.0 / jnp.sqrt(jnp.float32(1.0 + 1e-3))
    h1 = jnp.dot(z_ref[...], w1_ref[0:_ZDIM, :])
    h1 = h1 + jnp.dot(c_ref[...], w1_ref[_ZDIM:_ZDIM + _COND, :])
    h1 = b8_ref[1:2, :] * ((h1 + b8_ref[0:1, :]) * isq) + b8_ref[2:3, :]
    h1 = jnp.maximum(h1, 0.0)
    h2 = jnp.dot(h1, w2_ref[...])
    h2 = b8_ref[4:5, :] * ((h2 + b8_ref[3:4, :]) * isq) + b8_ref[5:6, :]
    h2 = jnp.maximum(h2, 0.0)
    t = jnp.dot(h2, wall_ref[...]) + ball_ref[...]
    a = jnp.tanh(t[:, _GRP:_WID])
    s = (t[:, :_GRP] + g_ref[...]) / _TAU
    sl = [s[:, k * _NGP:(k + 1) * _NGP] for k in range(_MODES)]
    m = sl[0]
    for k in range(1, _MODES):
        m = jnp.maximum(m, sl[k])
    es = [jnp.exp(v - m) for v in sl]
    den = es[0]
    for k in range(1, _MODES):
        den = den + es[k]
    ys = [e / den for e in es]
    ymax = ys[0]
    for k in range(1, _MODES):
        ymax = jnp.maximum(ymax, ys[k])
    outs = [((y == ymax).astype(jnp.float32) - y) + y for y in ys]
    u = jnp.concatenate(outs + [a], axis=1)
    fr = jnp.dot(u, p_ref[...])
    out_ref[...] = fr
    fr3 = fr.reshape(_BLK // _PAC, _PAC, _ROW)
    hc = jnp.dot(fr3[:, 0, :], wc1_ref[0:_ROW, :])
    for p in range(1, _PAC):
        hc = hc + jnp.dot(fr3[:, p, :], wc1_ref[p * _ROW:(p + 1) * _ROW, :])
    hc = hc + b8_ref[6:7, :]
    hc = jnp.where(hc >= 0, hc, 0.2 * hc)
    hc = jnp.dot(hc, wc2_ref[...]) + b8_ref[7:8, :]
    hc = jnp.where(hc >= 0, hc, 0.2 * hc)
    sc_ref[...] = jnp.dot(hc, wo_ref[...]) + b8_ref[8:9, 0:1]


def _critic_body(x_ref, w1_ref, b8_ref, w2_ref, wo_ref, out_ref):
    h = jnp.dot(x_ref[...], w1_ref[...]) + b8_ref[6:7, :]
    h = jnp.where(h >= 0, h, 0.2 * h)
    h = jnp.dot(h, w2_ref[...]) + b8_ref[7:8, :]
    h = jnp.where(h >= 0, h, 0.2 * h)
    out_ref[...] = jnp.dot(h, wo_ref[...]) + b8_ref[8:9, 0:1]


def _gumbel_noise() -> jnp.ndarray:
    try:
        with jax.ensure_compile_time_eval():
            return _gumbel_noise_impl()
    except Exception:
        return _gumbel_noise_impl()


def _gumbel_noise_impl() -> jnp.ndarray:
    gkey = jax.random.key(42)

    def draw(k):
        U = jax.random.uniform(k, (_B, _MODES), minval=1e-6, maxval=1.0 - 1e-6)
        return -jnp.log(-jnp.log(U + 1e-20) + 1e-20)

    kc = jax.vmap(lambda i: jax.random.fold_in(gkey, i))(jnp.arange(_N_CONT))
    kd = jax.vmap(lambda i: jax.random.fold_in(gkey, i))(jnp.arange(1000, 1000 + _N_DISC))
    gc = jax.vmap(draw)(kc)
    gd = jax.vmap(draw)(kd)
    g = jnp.concatenate([jnp.transpose(gc, (1, 2, 0)),
                         jnp.transpose(gd, (1, 2, 0)),
                         jnp.zeros((_B, _MODES, _NGP - _NG), jnp.float32)],
                        axis=2)
    return g.reshape(_B, _GRP)


def kernel(z, cond, W1, b1, g1, be1, W2, b2, g2, be2, Wa, ba, Wb, bb, Wd, bd,
           Wc1, bc1, Wc2, bc2, Wo, bo):
    wa = jnp.transpose(jnp.squeeze(Wa, -1))
    wg = jnp.concatenate([jnp.transpose(Wb, (1, 2, 0)),
                          jnp.transpose(Wd, (1, 2, 0)),
                          jnp.zeros((_HID, _MODES, _NGP - _NG), jnp.float32)],
                         axis=2)
    wall = jnp.concatenate([wg.reshape(_HID, _GRP), wa], axis=1)
    bgrp = jnp.concatenate([jnp.transpose(bb), jnp.transpose(bd),
                            jnp.zeros((_MODES, _NGP - _NG), jnp.float32)],
                           axis=1).reshape(_GRP)
    ball = jnp.concatenate([bgrp, jnp.squeeze(ba, -1)])[None, :]
    gfull = _gumbel_noise()
    pmat = jnp.asarray(_PERM_NP)
    b9 = jnp.stack([b1, g1, be1, b2, g2, be2, bc1, bc2,
                    jnp.concatenate([bo, jnp.zeros((_HID - 1,), jnp.float32)])])

    nblk = _B // _BLK
    full_row, score = pl.pallas_call(
        _gen_body,
        grid=(nblk,),
        in_specs=[
            pl.BlockSpec((_BLK, _ZDIM), lambda i: (i, 0)),
            pl.BlockSpec((_BLK, _COND), lambda i: (i, 0)),
            pl.BlockSpec((_ZDIM + _COND, _HID), lambda i: (0, 0)),
            pl.BlockSpec((9, _HID), lambda i: (0, 0)),
            pl.BlockSpec((_HID, _HID), lambda i: (0, 0)),
            pl.BlockSpec((_HID, _WID), lambda i: (0, 0)),
            pl.BlockSpec((1, _WID), lambda i: (0, 0)),
            pl.BlockSpec((_BLK, _GRP), lambda i: (i, 0)),
            pl.BlockSpec((_WID, _ROW), lambda i: (0, 0)),
            pl.BlockSpec((_PAC * _ROW, _HID), lambda i: (0, 0)),
            pl.BlockSpec((_HID, _HID), lambda i: (0, 0)),
            pl.BlockSpec((_HID, 1), lambda i: (0, 0)),
        ],
        out_specs=[
            pl.BlockSpec((_BLK, _ROW), lambda i: (i, 0)),
            pl.BlockSpec((_BLK // _PAC, 1), lambda i: (i, 0)),
        ],
        out_shape=[
            jax.ShapeDtypeStruct((_B, _ROW), jnp.float32),
            jax.ShapeDtypeStruct((_B // _PAC, 1), jnp.float32),
        ],
    )(z, cond, W1, b9, W2, wall, ball, gfull, pmat, Wc1, Wc2, Wo)
    return (full_row, jnp.squeeze(score, axis=1))
